# baseline (speedup 1.0000x reference)
import jax
import jax.numpy as jnp
from jax.experimental import pallas as pl
from jax.experimental.pallas import tpu as pltpu

_B, _C, _H, _W = 8, 16, 256, 128
_N = 1024
_NUMEL = _B * _C * _H * _W
_CBLK = 4
_NACC = 4
_NW = _CBLK * _W
_NBLK = _B * _C // _CBLK


def _vq_kernel(z_ref, w_ref, zq_ref, idx_ref, loss_ref,
               hwsq_ref, zcat_ref, pidx_ref):
    i = pl.program_id(0)

    @pl.when(i == 0)
    def _():
        w0 = w_ref[...]
        hwsq_ref[...] = 0.5 * jnp.sum(w0 * w0, axis=1, keepdims=True)
        loss_ref[...] = jnp.zeros_like(loss_ref)

    w = w_ref[...]

    @pl.when(i > 0)
    def _():
        pidx = pidx_ref[0, :]
        iiota = jax.lax.broadcasted_iota(jnp.int32, (_N, _NW), 0)
        onehot = jnp.where(iiota == pidx[None, :], 1.0, 0.0)
        zq = jax.lax.dot_general(
            w, onehot, (((0,), (0,)), ((), ())),
            preferred_element_type=jnp.float32,
            precision=jax.lax.Precision.DEFAULT)
        for c in range(_CBLK):
            zq_ref[0, c] = zq[:, c * _W:(c + 1) * _W]

    @pl.when(i < _NBLK)
    def _():
        for c in range(_CBLK):
            zcat_ref[:, c * _W:(c + 1) * _W] = z_ref[0, c]
        zcat = zcat_ref[...]
        m = jax.lax.dot_general(
            w, zcat, (((1,), (0,)), ((), ())),
            preferred_element_type=jnp.float32,
            precision=jax.lax.Precision.DEFAULT)
        neg = jnp.float32(-3.0e38)
        runmax = [jnp.full((8, _NW), neg, jnp.float32) for _ in range(_NACC)]
        runtile = [jnp.zeros((8, _NW), jnp.float32) for _ in range(_NACC)]
        for t in range(_N // 8):
            k = t % _NACC
            st = m[8 * t:8 * t + 8, :] - hwsq_ref[8 * t:8 * t + 8, :]
            gt = st > runmax[k]
            runtile[k] = jnp.where(gt, jnp.float32(t), runtile[k])
            runmax[k] = jnp.maximum(st, runmax[k])
        sub_i = jax.lax.broadcasted_iota(
            jnp.int32, (8, _NW), 0).astype(jnp.float32)
        vals = jnp.concatenate(runmax, axis=0)
        nidx = jnp.concatenate(
            [rt * 8.0 + sub_i for rt in runtile], axis=0)
        smax = jnp.max(vals, axis=0)
        idxf = jnp.min(
            jnp.where(vals == smax[None, :], nidx, jnp.float32(2 * _N)),
            axis=0)
        idx = idxf.astype(jnp.int32)
        pidx_ref[0, :] = idx
        for c in range(_CBLK):
            idx_ref[c, 0] = idx[c * _W:(c + 1) * _W]
        zsq = jnp.sum(zcat * zcat, axis=0)
        part = zsq - 2.0 * smax
        acc = part[0:_W]
        for c in range(1, _CBLK):
            acc = acc + part[c * _W:(c + 1) * _W]
        loss_ref[0:1, :] += acc[None, :]

    @pl.when(i == _NBLK)
    def _():
        total = jnp.sum(loss_ref[...]) * (1.25 / _NUMEL)
        loss_ref[...] = jnp.full((8, 128), total, jnp.float32)


def kernel(z, W):
    b, c, h, w = z.shape
    cpb = c // _CBLK
    last = _NBLK - 1
    zq, idx3, loss_arr = pl.pallas_call(
        _vq_kernel,
        grid=(_NBLK + 1,),
        in_specs=[
            pl.BlockSpec(
                (1, _CBLK, h, w),
                lambda i: (jnp.minimum(i, last) // cpb,
                           jnp.minimum(i, last) % cpb, 0, 0)),
            pl.BlockSpec((_N, h), lambda i: (0, 0)),
        ],
        out_specs=[
            pl.BlockSpec(
                (1, _CBLK, h, w),
                lambda i: (jnp.maximum(i - 1, 0) // cpb,
                           jnp.maximum(i - 1, 0) % cpb, 0, 0)),
            pl.BlockSpec((_CBLK, 1, w), lambda i: (jnp.minimum(i, last), 0, 0)),
            pl.BlockSpec((8, 128), lambda i: (0, 0)),
        ],
        out_shape=[
            jax.ShapeDtypeStruct((b, c, h, w), jnp.float32),
            jax.ShapeDtypeStruct((b * c, 1, w), jnp.int32),
            jax.ShapeDtypeStruct((8, 128), jnp.float32),
        ],
        scratch_shapes=[
            pltpu.VMEM((_N, 1), jnp.float32),
            pltpu.VMEM((h, _NW), jnp.float32),
            pltpu.VMEM((1, _NW), jnp.int32),
        ],
    )(z, W)
    return zq, loss_arr[0, 0], idx3.reshape(b, c, w)


# speedup vs baseline: 1.0111x; 1.0111x over previous
import jax
import jax.numpy as jnp
from jax.experimental import pallas as pl
from jax.experimental.pallas import tpu as pltpu

_B, _C, _H, _W = 8, 16, 256, 128
_N = 1024
_NUMEL = _B * _C * _H * _W
_CBLK = 4
_NACC = 8
_NW = _CBLK * _W


def _vq_kernel(z_ref, w_ref, zq_ref, idx_ref, loss_ref, hwsq_ref, zcat_ref):
    i = pl.program_id(0)

    @pl.when(i == 0)
    def _():
        w0 = w_ref[...]
        hwsq_ref[...] = 0.5 * jnp.sum(w0 * w0, axis=1, keepdims=True)
        loss_ref[...] = jnp.zeros_like(loss_ref)

    w = w_ref[...]
    for c in range(_CBLK):
        zcat_ref[:, c * _W:(c + 1) * _W] = z_ref[0, c]
    zcat = zcat_ref[...]
    m = jax.lax.dot_general(
        w, zcat, (((1,), (0,)), ((), ())),
        preferred_element_type=jnp.float32,
        precision=jax.lax.Precision.DEFAULT)
    neg = jnp.float32(-3.0e38)
    runmax = [jnp.full((8, _NW), neg, jnp.float32) for _ in range(_NACC)]
    runtile = [jnp.zeros((8, _NW), jnp.float32) for _ in range(_NACC)]
    for t in range(_N // 8):
        k = t % _NACC
        st = m[8 * t:8 * t + 8, :] - hwsq_ref[8 * t:8 * t + 8, :]
        gt = st > runmax[k]
        runtile[k] = jnp.where(gt, jnp.float32(t), runtile[k])
        runmax[k] = jnp.maximum(st, runmax[k])
    sub_i = jax.lax.broadcasted_iota(
        jnp.int32, (8, _NW), 0).astype(jnp.float32)
    vals = jnp.concatenate(runmax, axis=0)
    nidx = jnp.concatenate(
        [rt * 8.0 + sub_i for rt in runtile], axis=0)
    smax = jnp.max(vals, axis=0)
    idxf = jnp.min(
        jnp.where(vals == smax[None, :], nidx, jnp.float32(2 * _N)),
        axis=0)
    idx = idxf.astype(jnp.int32)
    iiota = jax.lax.broadcasted_iota(jnp.int32, (_N, _NW), 0)
    onehot = jnp.where(iiota == idx[None, :], 1.0, 0.0)
    zq = jax.lax.dot_general(
        w, onehot, (((0,), (0,)), ((), ())),
        preferred_element_type=jnp.float32,
        precision=jax.lax.Precision.DEFAULT)
    for c in range(_CBLK):
        zq_ref[0, c] = zq[:, c * _W:(c + 1) * _W]
        idx_ref[c, 0] = idx[c * _W:(c + 1) * _W]
    zsq = jnp.sum(zcat * zcat, axis=0)
    part = zsq - 2.0 * smax
    acc = part[0:_W]
    for c in range(1, _CBLK):
        acc = acc + part[c * _W:(c + 1) * _W]
    loss_ref[0:1, :] += acc[None, :]

    @pl.when(i == _B * _C // _CBLK - 1)
    def _():
        total = jnp.sum(loss_ref[...]) * (1.25 / _NUMEL)
        loss_ref[...] = jnp.full((8, 128), total, jnp.float32)


def kernel(z, W):
    b, c, h, w = z.shape
    nblk = (b * c) // _CBLK
    cpb = c // _CBLK
    zq, idx3, loss_arr = pl.pallas_call(
        _vq_kernel,
        grid=(nblk,),
        in_specs=[
            pl.BlockSpec((1, _CBLK, h, w), lambda i: (i // cpb, i % cpb, 0, 0)),
            pl.BlockSpec((_N, h), lambda i: (0, 0)),
        ],
        out_specs=[
            pl.BlockSpec((1, _CBLK, h, w), lambda i: (i // cpb, i % cpb, 0, 0)),
            pl.BlockSpec((_CBLK, 1, w), lambda i: (i, 0, 0)),
            pl.BlockSpec((8, 128), lambda i: (0, 0)),
        ],
        out_shape=[
            jax.ShapeDtypeStruct((b, c, h, w), jnp.float32),
            jax.ShapeDtypeStruct((b * c, 1, w), jnp.int32),
            jax.ShapeDtypeStruct((8, 128), jnp.float32),
        ],
        scratch_shapes=[
            pltpu.VMEM((_N, 1), jnp.float32),
            pltpu.VMEM((h, _NW), jnp.float32),
        ],
    )(z, W)
    return zq, loss_arr[0, 0], idx3.reshape(b, c, w)


# speedup vs baseline: 1.3634x; 1.3484x over previous
import jax
import jax.numpy as jnp
from jax.experimental import pallas as pl
from jax.experimental.pallas import tpu as pltpu

_B, _C, _H, _W = 8, 16, 256, 128
_N = 1024
_NUMEL = _B * _C * _H * _W
_CBLK = 8
_NACC = 4
_NW = _CBLK * _W


def _vq_kernel(z_ref, w_ref, zq_ref, idx_ref, loss_ref, hwsq_ref, zcat_ref):
    i = pl.program_id(0)

    @pl.when(i == 0)
    def _():
        w0 = w_ref[...]
        hwsq_ref[...] = 0.5 * jnp.sum(w0 * w0, axis=1, keepdims=True)
        loss_ref[...] = jnp.zeros_like(loss_ref)

    w = w_ref[...]
    for c in range(_CBLK):
        zcat_ref[:, c * _W:(c + 1) * _W] = z_ref[0, c]
    zcat = zcat_ref[...]
    m = jax.lax.dot_general(
        w, zcat, (((1,), (0,)), ((), ())),
        preferred_element_type=jnp.float32,
        precision=jax.lax.Precision.DEFAULT)
    neg = jnp.float32(-3.0e38)
    runmax = [jnp.full((8, _NW), neg, jnp.float32) for _ in range(_NACC)]
    runtile = [jnp.zeros((8, _NW), jnp.float32) for _ in range(_NACC)]
    for t in range(_N // 8):
        k = t % _NACC
        st = m[8 * t:8 * t + 8, :] - hwsq_ref[8 * t:8 * t + 8, :]
        gt = st > runmax[k]
        runtile[k] = jnp.where(gt, jnp.float32(t), runtile[k])
        runmax[k] = jnp.maximum(st, runmax[k])
    sub_i = jax.lax.broadcasted_iota(
        jnp.int32, (8, _NW), 0).astype(jnp.float32)
    vals = jnp.concatenate(runmax, axis=0)
    nidx = jnp.concatenate(
        [rt * 8.0 + sub_i for rt in runtile], axis=0)
    smax = jnp.max(vals, axis=0)
    idxf = jnp.min(
        jnp.where(vals == smax[None, :], nidx, jnp.float32(2 * _N)),
        axis=0)
    idx = idxf.astype(jnp.int32)
    iiota = jax.lax.broadcasted_iota(jnp.int32, (_N, _NW), 0)
    onehot = jnp.where(iiota == idx[None, :], 1.0, 0.0)
    zq = jax.lax.dot_general(
        w, onehot, (((0,), (0,)), ((), ())),
        preferred_element_type=jnp.float32,
        precision=jax.lax.Precision.DEFAULT)
    for c in range(_CBLK):
        zq_ref[0, c] = zq[:, c * _W:(c + 1) * _W]
        idx_ref[c, 0] = idx[c * _W:(c + 1) * _W]
    zsq = jnp.sum(zcat * zcat, axis=0)
    part = zsq - 2.0 * smax
    acc = part[0:_W]
    for c in range(1, _CBLK):
        acc = acc + part[c * _W:(c + 1) * _W]
    loss_ref[0:1, :] += acc[None, :]

    @pl.when(i == _B * _C // _CBLK - 1)
    def _():
        total = jnp.sum(loss_ref[...]) * (1.25 / _NUMEL)
        loss_ref[...] = jnp.full((8, 128), total, jnp.float32)


def kernel(z, W):
    b, c, h, w = z.shape
    nblk = (b * c) // _CBLK
    cpb = c // _CBLK
    zq, idx3, loss_arr = pl.pallas_call(
        _vq_kernel,
        grid=(nblk,),
        in_specs=[
            pl.BlockSpec((1, _CBLK, h, w), lambda i: (i // cpb, i % cpb, 0, 0)),
            pl.BlockSpec((_N, h), lambda i: (0, 0)),
        ],
        out_specs=[
            pl.BlockSpec((1, _CBLK, h, w), lambda i: (i // cpb, i % cpb, 0, 0)),
            pl.BlockSpec((_CBLK, 1, w), lambda i: (i, 0, 0)),
            pl.BlockSpec((8, 128), lambda i: (0, 0)),
        ],
        out_shape=[
            jax.ShapeDtypeStruct((b, c, h, w), jnp.float32),
            jax.ShapeDtypeStruct((b * c, 1, w), jnp.int32),
            jax.ShapeDtypeStruct((8, 128), jnp.float32),
        ],
        scratch_shapes=[
            pltpu.VMEM((_N, 1), jnp.float32),
            pltpu.VMEM((h, _NW), jnp.float32),
        ],
    )(z, W)
    return zq, loss_arr[0, 0], idx3.reshape(b, c, w)


# speedup vs baseline: 1.5776x; 1.1571x over previous
import jax
import jax.numpy as jnp
from jax.experimental import pallas as pl
from jax.experimental.pallas import tpu as pltpu

_B, _C, _H, _W = 8, 16, 256, 128
_N = 1024
_NUMEL = _B * _C * _H * _W
_CBLK = 16
_NACC = 4
_NW = _CBLK * _W


def _vq_kernel(z_ref, w_ref, zq_ref, idx_ref, loss_ref, hwsq_ref, zcat_ref):
    i = pl.program_id(0)

    @pl.when(i == 0)
    def _():
        w0 = w_ref[...]
        hwsq_ref[...] = 0.5 * jnp.sum(w0 * w0, axis=1, keepdims=True)
        loss_ref[...] = jnp.zeros_like(loss_ref)

    w = w_ref[...]
    for c in range(_CBLK):
        zcat_ref[:, c * _W:(c + 1) * _W] = z_ref[0, c]
    zcat = zcat_ref[...]
    m = jax.lax.dot_general(
        w, zcat, (((1,), (0,)), ((), ())),
        preferred_element_type=jnp.float32,
        precision=jax.lax.Precision.DEFAULT)
    neg = jnp.float32(-3.0e38)
    runmax = [jnp.full((8, _NW), neg, jnp.float32) for _ in range(_NACC)]
    runtile = [jnp.zeros((8, _NW), jnp.float32) for _ in range(_NACC)]
    for t in range(_N // 8):
        k = t % _NACC
        st = m[8 * t:8 * t + 8, :] - hwsq_ref[8 * t:8 * t + 8, :]
        gt = st > runmax[k]
        runtile[k] = jnp.where(gt, jnp.float32(t), runtile[k])
        runmax[k] = jnp.maximum(st, runmax[k])
    sub_i = jax.lax.broadcasted_iota(
        jnp.int32, (8, _NW), 0).astype(jnp.float32)
    vals = jnp.concatenate(runmax, axis=0)
    nidx = jnp.concatenate(
        [rt * 8.0 + sub_i for rt in runtile], axis=0)
    smax = jnp.max(vals, axis=0)
    idxf = jnp.min(
        jnp.where(vals == smax[None, :], nidx, jnp.float32(2 * _N)),
        axis=0)
    idx = idxf.astype(jnp.int32)
    iiota = jax.lax.broadcasted_iota(jnp.int32, (_N, _NW), 0)
    onehot = jnp.where(iiota == idx[None, :], 1.0, 0.0)
    zq = jax.lax.dot_general(
        w, onehot, (((0,), (0,)), ((), ())),
        preferred_element_type=jnp.float32,
        precision=jax.lax.Precision.DEFAULT)
    for c in range(_CBLK):
        zq_ref[0, c] = zq[:, c * _W:(c + 1) * _W]
        idx_ref[c, 0] = idx[c * _W:(c + 1) * _W]
    zsq = jnp.sum(zcat * zcat, axis=0)
    part = zsq - 2.0 * smax
    acc = part[0:_W]
    for c in range(1, _CBLK):
        acc = acc + part[c * _W:(c + 1) * _W]
    loss_ref[0:1, :] += acc[None, :]

    @pl.when(i == _B * _C // _CBLK - 1)
    def _():
        total = jnp.sum(loss_ref[...]) * (1.25 / _NUMEL)
        loss_ref[...] = jnp.full((8, 128), total, jnp.float32)


def kernel(z, W):
    b, c, h, w = z.shape
    nblk = (b * c) // _CBLK
    cpb = c // _CBLK
    zq, idx3, loss_arr = pl.pallas_call(
        _vq_kernel,
        grid=(nblk,),
        in_specs=[
            pl.BlockSpec((1, _CBLK, h, w), lambda i: (i // cpb, i % cpb, 0, 0)),
            pl.BlockSpec((_N, h), lambda i: (0, 0)),
        ],
        out_specs=[
            pl.BlockSpec((1, _CBLK, h, w), lambda i: (i // cpb, i % cpb, 0, 0)),
            pl.BlockSpec((_CBLK, 1, w), lambda i: (i, 0, 0)),
            pl.BlockSpec((8, 128), lambda i: (0, 0)),
        ],
        out_shape=[
            jax.ShapeDtypeStruct((b, c, h, w), jnp.float32),
            jax.ShapeDtypeStruct((b * c, 1, w), jnp.int32),
            jax.ShapeDtypeStruct((8, 128), jnp.float32),
        ],
        scratch_shapes=[
            pltpu.VMEM((_N, 1), jnp.float32),
            pltpu.VMEM((h, _NW), jnp.float32),
        ],
    )(z, W)
    return zq, loss_arr[0, 0], idx3.reshape(b, c, w)
